# Initial kernel scaffold; baseline (speedup 1.0000x reference)
import jax
import jax.numpy as jnp
from jax.experimental import pallas as pl


def kernel(x, W1, g1, b1, W2, g2, b2, W3, g3, b3, W4, g4, b4, W5, g5, b5, L1, g6, b6, L2, bL2, g7, b7, Wc, bc):
    raise NotImplementedError("write your pallas kernel here")


# speedup vs baseline: 6.9775x; 6.9775x over previous
import functools

import jax
import jax.numpy as jnp
from jax import lax
from jax.experimental import pallas as pl
from jax.experimental.pallas import tpu as pltpu
from jax.experimental.pallas import tpu_sc as plsc

B = 8
N = 1024
KNN = 20
BLK = 256
HI = lax.Precision.HIGHEST
F32 = jnp.float32
BF16 = jnp.bfloat16
NEG = -3.4e38


def _knn_body(h_ref, idx_ref):
    b = pl.program_id(0)
    i = pl.program_id(1)
    h = h_ref[0]
    hblk = h_ref[0, pl.ds(i * BLK, BLK), :]
    xx = jnp.sum(h * h, axis=1)
    xxb = jnp.sum(hblk * hblk, axis=1)
    dots = lax.dot_general(hblk.astype(BF16), h.astype(BF16),
                           (((1,), (1,)), ((), ())),
                           preferred_element_type=F32)
    inner = -2.0 * dots
    pd = (-xxb[:, None] - inner) - xx[None, :]
    iota = lax.broadcasted_iota(jnp.int32, (BLK, N), 1)
    cols = []
    for _ in range(KNN):
        val = jnp.max(pd, axis=1, keepdims=True)
        sel = jnp.min(jnp.where(pd == val, iota, N), axis=1, keepdims=True)
        cols.append(sel)
        pd = jnp.where(iota == sel, NEG, pd)
    idx_ref[0] = jnp.concatenate(cols, axis=1) + b * N


def _knn(h):
    C = h.shape[2]
    return pl.pallas_call(
        _knn_body,
        grid=(B, N // BLK),
        in_specs=[pl.BlockSpec((1, N, C), lambda b, i: (b, 0, 0))],
        out_specs=pl.BlockSpec((1, BLK, KNN), lambda b, i: (b, i, 0)),
        out_shape=jax.ShapeDtypeStruct((B, N, KNN), jnp.int32),
    )(h)


@functools.lru_cache(maxsize=None)
def _gather_builder(CT):
    info = plsc.get_sparse_core_info()
    NC, NS = info.num_cores, info.num_subcores
    NW = NC * NS
    ROWS = KNN * B * N
    RPW = ROWS // NW
    IB = 128
    NBLK = RPW // IB
    mesh = plsc.VectorSubcoreMesh(core_axis_name="c", subcore_axis_name="s")

    @functools.partial(
        pl.kernel,
        mesh=mesh,
        out_type=jax.ShapeDtypeStruct((ROWS, CT), F32),
        scratch_types=[
            pltpu.VMEM((NBLK, IB), jnp.int32),
            pltpu.VMEM((IB, CT), F32),
            pltpu.VMEM((IB, CT), F32),
            pltpu.SemaphoreType.DMA,
        ],
    )
    def gather(h_hbm, idx_hbm, out_hbm, idx_v, rows_a, rows_b, sem):
        wid = lax.axis_index("s") * NC + lax.axis_index("c")
        pltpu.sync_copy(idx_hbm.at[wid], idx_v)
        base = wid * NBLK

        def seq(i, carry):
            pltpu.async_copy(h_hbm.at[idx_v.at[i]], rows_a, sem).wait()
            pltpu.sync_copy(rows_a, out_hbm.at[pl.ds((base + i) * IB, IB)])
            return carry

        lax.fori_loop(0, NBLK, seq, 0)

    return gather, NW, NBLK, IB


def _gather(h_pad, idx_t):
    CT = h_pad.shape[1]
    fn, NW, NBLK, IB = _gather_builder(CT)
    idx_r = idx_t.reshape(NW, NBLK, IB)
    return fn(h_pad, idx_r)


PB = 512


def _tree_sum(x):
    rows = x.shape[0]
    while rows > 8 and rows % 2 == 0:
        half = rows // 2
        x = x[:half] + x[half:]
        rows = half
    return jnp.sum(x, axis=0, keepdims=True)


def _comp_tree_sum(x):
    comp = x * 0.0
    rows = x.shape[0]
    while rows > 2 and rows % 2 == 0:
        half = rows // 2
        a = x[:half]
        b = x[half:]
        t = a + b
        bv = t - a
        e = (a - (t - bv)) + (b - bv)
        comp = comp[:half] + comp[half:] + e
        x = t
        rows = half
    return (jnp.sum(x, axis=0, keepdims=True)
            + jnp.sum(comp, axis=0, keepdims=True))


def _edge_tc_builder(CR, CE, CT, O):
    NPB = (B * N) // PB

    def body(g_ref, h_ref, w_ref, mx_ref, mn_ref, s_ref):
        pb = pl.program_id(0)
        j = pl.program_id(1)
        g = g_ref[:, :CR]
        h = h_ref[:, :CR]
        ec = jnp.concatenate([g - h, h], axis=1)
        f = lax.dot_general(ec.astype(BF16), w_ref[...].astype(BF16),
                            (((1,), (0,)), ((), ())),
                            preferred_element_type=F32)

        @pl.when(j == 0)
        def _():
            mx_ref[...] = f
            mn_ref[...] = f

        @pl.when(j != 0)
        def _():
            mx_ref[...] = jnp.maximum(mx_ref[...], f)
            mn_ref[...] = jnp.minimum(mn_ref[...], f)

        s = _comp_tree_sum(f)
        dc = f - s / jnp.float32(PB)
        s_ref[0] = jnp.concatenate([s, _tree_sum(dc * dc)], axis=0)

    return pl.pallas_call(
        body,
        grid=(NPB, KNN),
        in_specs=[
            pl.BlockSpec((PB, CT), lambda pb, j: (j * NPB + pb, 0)),
            pl.BlockSpec((PB, CE), lambda pb, j: (pb, 0)),
            pl.BlockSpec((2 * CR, O), lambda pb, j: (0, 0)),
        ],
        out_specs=[
            pl.BlockSpec((PB, O), lambda pb, j: (pb, 0)),
            pl.BlockSpec((PB, O), lambda pb, j: (pb, 0)),
            pl.BlockSpec((1, 2, O), lambda pb, j: (pb * KNN + j, 0, 0)),
        ],
        out_shape=[
            jax.ShapeDtypeStruct((B * N, O), F32),
            jax.ShapeDtypeStruct((B * N, O), F32),
            jax.ShapeDtypeStruct((NPB * KNN, 2, O), F32),
        ],
    )


def _edge_tc(G, h_flat, wt):
    CT = G.shape[1]
    CE = h_flat.shape[1]
    CR = wt.shape[0] // 2
    O = wt.shape[1]
    return _edge_tc_builder(CR, CE, CT, O)(G, h_flat, wt)


def _stats_body(s_ref, mv_ref):
    T = jnp.float32(B * N * KNN)
    parts = s_ref[...]
    sall = parts[:, 0, :]
    m2all = parts[:, 1, :]
    m = _comp_tree_sum(sall) / T
    dm = sall / jnp.float32(PB) - m
    var = (_tree_sum(m2all)
           + jnp.float32(PB) * _tree_sum(dm * dm)) / T
    mv_ref[...] = jnp.concatenate([m, var], axis=0)


def _stats(sums):
    O = sums.shape[2]
    return pl.pallas_call(
        _stats_body,
        out_shape=jax.ShapeDtypeStruct((2, O), F32),
    )(sums)


def _apply_body(mx_ref, mn_ref, mv_ref, g_ref, b_ref, out_ref):
    m = mv_ref[0:1, :]
    var = mv_ref[1:2, :]
    g = g_ref[...]
    sel = jnp.where(g >= 0.0, mx_ref[...], mn_ref[...])
    u = (sel - m) / jnp.sqrt(var + 1e-5) * g + b_ref[...]
    out_ref[...] = jnp.where(u >= 0.0, u, 0.2 * u)


def _apply(mx, mn, mv, g, b):
    O = mx.shape[1]
    RB = 1024
    return pl.pallas_call(
        _apply_body,
        grid=((B * N) // RB,),
        in_specs=[
            pl.BlockSpec((RB, O), lambda r: (r, 0)),
            pl.BlockSpec((RB, O), lambda r: (r, 0)),
            pl.BlockSpec((2, O), lambda r: (0, 0)),
            pl.BlockSpec((1, O), lambda r: (0, 0)),
            pl.BlockSpec((1, O), lambda r: (0, 0)),
        ],
        out_specs=pl.BlockSpec((RB, O), lambda r: (r, 0)),
        out_shape=jax.ShapeDtypeStruct((B * N, O), F32),
    )(mx, mn, mv, g.reshape(1, O), b.reshape(1, O))


def _conv5_body(h1_ref, h2_ref, h3_ref, h4_ref, w_ref, y_ref, sums_ref):
    b = pl.program_id(0)
    i = pl.program_id(1)
    acc = lax.dot_general(h1_ref[0].astype(BF16),
                          w_ref[pl.ds(0, 64), :].astype(BF16),
                          (((1,), (0,)), ((), ())),
                          preferred_element_type=F32)
    acc += lax.dot_general(h2_ref[0].astype(BF16),
                           w_ref[pl.ds(64, 64), :].astype(BF16),
                           (((1,), (0,)), ((), ())),
                           preferred_element_type=F32)
    acc += lax.dot_general(h3_ref[0].astype(BF16),
                           w_ref[pl.ds(128, 128), :].astype(BF16),
                           (((1,), (0,)), ((), ())),
                           preferred_element_type=F32)
    acc += lax.dot_general(h4_ref[0].astype(BF16),
                           w_ref[pl.ds(256, 256), :].astype(BF16),
                           (((1,), (0,)), ((), ())),
                           preferred_element_type=F32)
    y_ref[0] = acc
    s = jnp.sum(acc, axis=0, keepdims=True)
    s2 = jnp.sum(acc * acc, axis=0, keepdims=True)
    st = jnp.concatenate([s, s2], axis=0)
    first = jnp.logical_and(b == 0, i == 0)

    @pl.when(first)
    def _():
        sums_ref[...] = st

    @pl.when(jnp.logical_not(first))
    def _():
        sums_ref[...] = sums_ref[...] + st


def _conv5(h1, h2, h3, h4, w5_t):
    return pl.pallas_call(
        _conv5_body,
        grid=(B, N // BLK),
        in_specs=[
            pl.BlockSpec((1, BLK, 64), lambda b, i: (b, i, 0)),
            pl.BlockSpec((1, BLK, 64), lambda b, i: (b, i, 0)),
            pl.BlockSpec((1, BLK, 128), lambda b, i: (b, i, 0)),
            pl.BlockSpec((1, BLK, 256), lambda b, i: (b, i, 0)),
            pl.BlockSpec((512, 1024), lambda b, i: (0, 0)),
        ],
        out_specs=[
            pl.BlockSpec((1, BLK, 1024), lambda b, i: (b, i, 0)),
            pl.BlockSpec((2, 1024), lambda b, i: (0, 0)),
        ],
        out_shape=[
            jax.ShapeDtypeStruct((B, N, 1024), F32),
            jax.ShapeDtypeStruct((2, 1024), F32),
        ],
    )(h1, h2, h3, h4, w5_t)


def _pool_body(y_ref, sums_ref, g_ref, b_ref, y1_ref, y2_ref):
    i = pl.program_id(1)
    T = jnp.float32(B * N)
    m = sums_ref[0:1, :] / T
    var = sums_ref[1:2, :] / T - m * m
    scale = g_ref[...] / jnp.sqrt(var + 1e-5)
    shift = b_ref[...] - m * scale
    u = y_ref[0] * scale + shift
    lr = jnp.where(u >= 0.0, u, 0.2 * u)
    mx = jnp.max(lr, axis=0, keepdims=True)
    sm = jnp.sum(lr, axis=0, keepdims=True) / jnp.float32(N)

    @pl.when(i == 0)
    def _():
        y1_ref[0] = mx
        y2_ref[0] = sm

    @pl.when(i != 0)
    def _():
        y1_ref[0] = jnp.maximum(y1_ref[0], mx)
        y2_ref[0] = y2_ref[0] + sm


def _pool(y, sums, g5, b5):
    return pl.pallas_call(
        _pool_body,
        grid=(B, N // BLK),
        in_specs=[
            pl.BlockSpec((1, BLK, 1024), lambda b, i: (b, i, 0)),
            pl.BlockSpec((2, 1024), lambda b, i: (0, 0)),
            pl.BlockSpec((1, 1024), lambda b, i: (0, 0)),
            pl.BlockSpec((1, 1024), lambda b, i: (0, 0)),
        ],
        out_specs=[
            pl.BlockSpec((1, 1, 1024), lambda b, i: (b, 0, 0)),
            pl.BlockSpec((1, 1, 1024), lambda b, i: (b, 0, 0)),
        ],
        out_shape=[
            jax.ShapeDtypeStruct((B, 1, 1024), F32),
            jax.ShapeDtypeStruct((B, 1, 1024), F32),
        ],
    )(y, sums, g5.reshape(1, 1024), b5.reshape(1, 1024))


def _head_body(y1_ref, y2_ref, l1_ref, g6_ref, b6_ref, l2_ref, bl2_ref,
               g7_ref, b7_ref, wc_ref, bc_ref, out_ref):
    z = jnp.concatenate([y1_ref[...], y2_ref[...]], axis=1)
    a = lax.dot_general(z.astype(BF16), l1_ref[...].astype(BF16),
                        (((1,), (0,)), ((), ())),
                        preferred_element_type=F32)
    m = jnp.mean(a, axis=0, keepdims=True)
    v = jnp.mean((a - m) * (a - m), axis=0, keepdims=True)
    a = (a - m) / jnp.sqrt(v + 1e-5) * g6_ref[...] + b6_ref[...]
    a = jnp.where(a >= 0.0, a, 0.2 * a)
    a2 = lax.dot_general(a.astype(BF16), l2_ref[...].astype(BF16),
                         (((1,), (0,)), ((), ())),
                         preferred_element_type=F32) + bl2_ref[...]
    m2 = jnp.mean(a2, axis=0, keepdims=True)
    v2 = jnp.mean((a2 - m2) * (a2 - m2), axis=0, keepdims=True)
    a2 = (a2 - m2) / jnp.sqrt(v2 + 1e-5) * g7_ref[...] + b7_ref[...]
    a2 = jnp.where(a2 >= 0.0, a2, 0.2 * a2)
    out_ref[...] = lax.dot_general(a2.astype(BF16), wc_ref[...].astype(BF16),
                                   (((1,), (0,)), ((), ())),
                                   preferred_element_type=F32) + bc_ref[...]


def _head(y1, y2, l1_t, g6, b6, l2_t, bl2, g7, b7, wc_t, bc):
    return pl.pallas_call(
        _head_body,
        out_shape=jax.ShapeDtypeStruct((B, 40), F32),
    )(y1, y2, l1_t, g6.reshape(1, 512), b6.reshape(1, 512), l2_t,
      bl2.reshape(1, 256), g7.reshape(1, 256), b7.reshape(1, 256), wc_t,
      bc.reshape(1, 40))


def _edge_conv(h, W, g, b):
    C = h.shape[2]
    O = W.shape[0]
    if C == 3:
        h = jnp.pad(h, ((0, 0), (0, 0), (0, 5)))
        CE = 8
    else:
        CE = C
    wt = W.T
    idx = _knn(h)
    idx_t = jnp.transpose(idx, (2, 0, 1)).reshape(KNN * B * N)
    h_flat = h.reshape(B * N, CE)
    CT = max(CE, 128)
    h_pad = jnp.pad(h_flat, ((0, 0), (0, CT - CE))) if CT != CE else h_flat
    G = _gather(h_pad, idx_t)
    mx, mn, sums = _edge_tc(G, h_flat, wt)
    mv = _stats(sums)
    return _apply(mx, mn, mv, g, b).reshape(B, N, O)


def kernel(x, W1, g1, b1, W2, g2, b2, W3, g3, b3, W4, g4, b4, W5, g5, b5,
           L1, g6, b6, L2, bL2, g7, b7, Wc, bc):
    h0 = jnp.transpose(x, (0, 2, 1))
    h1 = _edge_conv(h0, W1, g1, b1)
    h2 = _edge_conv(h1, W2, g2, b2)
    h3 = _edge_conv(h2, W3, g3, b3)
    h4 = _edge_conv(h3, W4, g4, b4)
    y, sums = _conv5(h1.reshape(B, N, 64), h2.reshape(B, N, 64),
                     h3.reshape(B, N, 128), h4.reshape(B, N, 256), W5.T)
    y1, y2 = _pool(y, sums, g5, b5)
    return _head(y1.reshape(B, 1024), y2.reshape(B, 1024), L1.T, g6, b6,
                 L2.T, bL2, g7, b7, Wc.T, bc)
